# baseline (speedup 1.0000x reference)
import jax
import jax.numpy as jnp
from jax.experimental import pallas as pl

S, B, W = 20, 1024, 21
V, CE, F = 96, 50, 128
K = 5
T = W - K + 1
N = S * B
NB = 512
VP = 128


def _fused_body(idx_ref, w3_ref, cb_ref, wp_ref, bp_ref, wg_ref,
                bg_ref, out_ref):
    idx = idx_ref[...]
    iot = jax.lax.broadcasted_iota(jnp.int32, (W, NB, VP), 2)
    oh = (idx[:, :, None] == iot).astype(jnp.bfloat16)
    oh2 = oh.reshape(W * NB, VP)
    ohc = jnp.concatenate([oh2[k * NB:(k + T) * NB] for k in range(K)],
                          axis=1)
    acc = jnp.dot(ohc, w3_ref[...], preferred_element_type=jnp.float32)
    m = jnp.maximum(jnp.max(acc.reshape(T, NB, F), axis=0) + cb_ref[...],
                    0.0)
    hp = jnp.maximum(
        jnp.dot(m, wp_ref[...], preferred_element_type=jnp.float32)
        + bp_ref[...], 0.0)
    hg = jax.nn.sigmoid(
        jnp.dot(m, wg_ref[...], preferred_element_type=jnp.float32)
        + bg_ref[...])
    out_ref[...] = hg * hp + (1.0 - hg) * m


def kernel(input, char_emb, conv_w, conv_b, w_proj, b_proj, w_gate, b_gate):
    idxp = input.reshape(N, W).T
    w3 = jnp.einsum('vc,fck->kvf', char_emb, conv_w)
    w3 = (jnp.zeros((K, VP, F), jnp.float32).at[:, :V, :].set(w3)
          .reshape(K * VP, F).astype(jnp.bfloat16))
    cb2 = conv_b.reshape(1, F)
    bp2 = b_proj.reshape(1, F)
    bg2 = b_gate.reshape(1, F)

    out = pl.pallas_call(
        _fused_body,
        grid=(N // NB,),
        in_specs=[
            pl.BlockSpec((W, NB), lambda i: (0, i)),
            pl.BlockSpec((K * VP, F), lambda i: (0, 0)),
            pl.BlockSpec((1, F), lambda i: (0, 0)),
            pl.BlockSpec((F, F), lambda i: (0, 0)),
            pl.BlockSpec((1, F), lambda i: (0, 0)),
            pl.BlockSpec((F, F), lambda i: (0, 0)),
            pl.BlockSpec((1, F), lambda i: (0, 0)),
        ],
        out_specs=pl.BlockSpec((NB, F), lambda i: (i, 0)),
        out_shape=jax.ShapeDtypeStruct((N, F), jnp.float32),
    )(idxp, w3, cb2, w_proj.T, bp2, w_gate.T, bg2)
    return out.reshape(S, B, F)


# speedup vs baseline: 1.0475x; 1.0475x over previous
import jax
import jax.numpy as jnp
from jax.experimental import pallas as pl

S, B, W = 20, 1024, 21
V, CE, F = 96, 50, 128
K = 5
T = W - K + 1
N = S * B
NB = 1024
VP = 128


def _fused_body(idx_ref, w3_ref, cb_ref, wp_ref, bp_ref, wg_ref,
                bg_ref, out_ref):
    idx = idx_ref[...]
    iot = jax.lax.broadcasted_iota(jnp.int32, (W, NB, VP), 2)
    oh = (idx[:, :, None] == iot).astype(jnp.bfloat16)
    oh2 = oh.reshape(W * NB, VP)
    ohc = jnp.concatenate([oh2[k * NB:(k + T) * NB] for k in range(K)],
                          axis=1)
    acc = jnp.dot(ohc, w3_ref[...], preferred_element_type=jnp.float32)
    m = jnp.maximum(jnp.max(acc.reshape(T, NB, F), axis=0) + cb_ref[...],
                    0.0)
    hp = jnp.maximum(
        jnp.dot(m, wp_ref[...], preferred_element_type=jnp.float32)
        + bp_ref[...], 0.0)
    hg = jax.nn.sigmoid(
        jnp.dot(m, wg_ref[...], preferred_element_type=jnp.float32)
        + bg_ref[...])
    out_ref[...] = hg * hp + (1.0 - hg) * m


def kernel(input, char_emb, conv_w, conv_b, w_proj, b_proj, w_gate, b_gate):
    idxp = input.reshape(N, W).T
    w3 = jnp.einsum('vc,fck->kvf', char_emb, conv_w)
    w3 = (jnp.zeros((K, VP, F), jnp.float32).at[:, :V, :].set(w3)
          .reshape(K * VP, F).astype(jnp.bfloat16))
    cb2 = conv_b.reshape(1, F)
    bp2 = b_proj.reshape(1, F)
    bg2 = b_gate.reshape(1, F)

    out = pl.pallas_call(
        _fused_body,
        grid=(N // NB,),
        in_specs=[
            pl.BlockSpec((W, NB), lambda i: (0, i)),
            pl.BlockSpec((K * VP, F), lambda i: (0, 0)),
            pl.BlockSpec((1, F), lambda i: (0, 0)),
            pl.BlockSpec((F, F), lambda i: (0, 0)),
            pl.BlockSpec((1, F), lambda i: (0, 0)),
            pl.BlockSpec((F, F), lambda i: (0, 0)),
            pl.BlockSpec((1, F), lambda i: (0, 0)),
        ],
        out_specs=pl.BlockSpec((NB, F), lambda i: (i, 0)),
        out_shape=jax.ShapeDtypeStruct((N, F), jnp.float32),
    )(idxp, w3, cb2, w_proj.T, bp2, w_gate.T, bg2)
    return out.reshape(S, B, F)


# speedup vs baseline: 1.0672x; 1.0188x over previous
import jax
import jax.numpy as jnp
from jax.experimental import pallas as pl

S, B, W = 20, 1024, 21
V, CE, F = 96, 50, 128
K = 5
T = W - K + 1
N = S * B
NB = 2048
VP = 128


def _fused_body(idx_ref, w3_ref, cb_ref, wp_ref, bp_ref, wg_ref,
                bg_ref, out_ref):
    idx = idx_ref[...]
    iot = jax.lax.broadcasted_iota(jnp.int32, (W, NB, VP), 2)
    oh = (idx[:, :, None] == iot).astype(jnp.bfloat16)
    oh2 = oh.reshape(W * NB, VP)
    ohc = jnp.concatenate([oh2[k * NB:(k + T) * NB] for k in range(K)],
                          axis=1)
    acc = jnp.dot(ohc, w3_ref[...], preferred_element_type=jnp.float32)
    m = jnp.maximum(jnp.max(acc.reshape(T, NB, F), axis=0) + cb_ref[...],
                    0.0)
    hp = jnp.maximum(
        jnp.dot(m, wp_ref[...], preferred_element_type=jnp.float32)
        + bp_ref[...], 0.0)
    hg = jax.nn.sigmoid(
        jnp.dot(m, wg_ref[...], preferred_element_type=jnp.float32)
        + bg_ref[...])
    out_ref[...] = hg * hp + (1.0 - hg) * m


def kernel(input, char_emb, conv_w, conv_b, w_proj, b_proj, w_gate, b_gate):
    idxp = input.reshape(N, W).T
    w3 = jnp.einsum('vc,fck->kvf', char_emb, conv_w)
    w3 = (jnp.zeros((K, VP, F), jnp.float32).at[:, :V, :].set(w3)
          .reshape(K * VP, F).astype(jnp.bfloat16))
    cb2 = conv_b.reshape(1, F)
    bp2 = b_proj.reshape(1, F)
    bg2 = b_gate.reshape(1, F)

    out = pl.pallas_call(
        _fused_body,
        grid=(N // NB,),
        in_specs=[
            pl.BlockSpec((W, NB), lambda i: (0, i)),
            pl.BlockSpec((K * VP, F), lambda i: (0, 0)),
            pl.BlockSpec((1, F), lambda i: (0, 0)),
            pl.BlockSpec((F, F), lambda i: (0, 0)),
            pl.BlockSpec((1, F), lambda i: (0, 0)),
            pl.BlockSpec((F, F), lambda i: (0, 0)),
            pl.BlockSpec((1, F), lambda i: (0, 0)),
        ],
        out_specs=pl.BlockSpec((NB, F), lambda i: (i, 0)),
        out_shape=jax.ShapeDtypeStruct((N, F), jnp.float32),
    )(idxp, w3, cb2, w_proj.T, bp2, w_gate.T, bg2)
    return out.reshape(S, B, F)
